# baseline (speedup 1.0000x reference)
import jax
import jax.numpy as jnp
from jax.experimental import pallas as pl

N, E, C, H = 10000, 160000, 128, 128


def kernel(x, edge_index, Wg0, bg0, lng0, lnb0, Ws1_0, bs1_0, Ws2_0, bs2_0, Wg1, bg1, lng1, lnb1, Ws1_1, bs1_1, Ws2_1, bs2_1, Wg2, bg2, lng2, lnb2, Ws1_2, bs1_2, Ws2_2, bs2_2, Wa, ba, Wp1, bp1, Wp2, bp2, Wp3, bp3):
    p = {
        'Wg0': Wg0, 'bg0': bg0, 'lng0': lng0, 'lnb0': lnb0,
        'Ws1_0': Ws1_0, 'bs1_0': bs1_0, 'Ws2_0': Ws2_0, 'bs2_0': bs2_0,
        'Wg1': Wg1, 'bg1': bg1, 'lng1': lng1, 'lnb1': lnb1,
        'Ws1_1': Ws1_1, 'bs1_1': bs1_1, 'Ws2_1': Ws2_1, 'bs2_1': bs2_1,
        'Wg2': Wg2, 'bg2': bg2, 'lng2': lng2, 'lnb2': lnb2,
        'Ws1_2': Ws1_2, 'bs1_2': bs1_2, 'Ws2_2': Ws2_2, 'bs2_2': bs2_2,
    }
    src = edge_index[0]
    dst = edge_index[1]
    base = jax.random.key(42)
    h = x
    feats = []
    for l in range(3):
        Wg, bg = p['Wg%d' % l], p['bg%d' % l]
        lng, lnb = p['lng%d' % l], p['lnb%d' % l]
        Ws1, bs1 = p['Ws1_%d' % l], p['bs1_%d' % l]
        Ws2, bs2 = p['Ws2_%d' % l], p['bs2_%d' % l]
        u = jax.random.uniform(jax.random.fold_in(base, l), (E,), jnp.float32)
        t = jnp.log(u) - jnp.log1p(-u)
        WA = lng[:C, None] * Ws1[:C]
        WB = lng[C:, None] * Ws1[C:]
        g64 = lng @ Ws1
        c0 = lnb @ Ws1 + bs1
        xw = h @ Wg
        A = h @ WA
        B = h @ WB
        S = h.sum(1)
        Q = (h * h).sum(1)
        mu = (S[src] + S[dst]) / (2 * C)
        var = (Q[src] + Q[dst]) / (2 * C) - mu * mu
        rs = jax.lax.rsqrt(var + 1e-5)
        v = (A[src] + B[dst]) * rs[:, None] + (c0[None, :] - (mu * rs)[:, None] * g64[None, :])
        z = jax.nn.relu(v) @ Ws2[:, 0] + bs2[0]
        mask = (z > t).astype(jnp.float32)
        deg = jnp.zeros((N,), jnp.float32).at[dst].add(mask) + 1.0
        dis = jax.lax.rsqrt(deg)
        y = xw * dis[:, None]
        acc = jnp.zeros((N, H), jnp.float32).at[dst].add(y[src] * mask[:, None])
        out = acc * dis[:, None] + xw * (dis * dis)[:, None] + bg
        h = out
        if l != 2:
            h = jax.nn.relu(h)
            feats.append(h)
    aw = jnp.concatenate([jax.nn.sigmoid(f @ Wa + ba) for f in feats], axis=1)
    aw = jax.nn.softmax(aw, axis=1)
    ms = aw[:, 0:1] * feats[0] + aw[:, 1:2] * feats[1]
    h = h + ms
    z = jax.nn.relu(h @ Wp1 + bp1)
    z = jax.nn.relu(z @ Wp2 + bp2)
    z = z @ Wp3 + bp3
    return jax.nn.sigmoid(z)


# speedup vs baseline: 7.5155x; 7.5155x over previous
import functools

import jax
import jax.numpy as jnp
from jax import lax
from jax.experimental import pallas as pl
from jax.experimental.pallas import tpu as pltpu
from jax.experimental.pallas import tpu_sc as plsc

N = 10000
NP = 10240
E = 160000
C = 128
HD = 128
DT = 128
NC = 2
NS = 16
NW = NC * NS
EPT = E // NW
CHW = 128
NCHK = 40
EPT_PAD = NCHK * CHW

_f32 = jnp.float32
_i32 = jnp.int32


_BN = 1024


def _tc_pre_body(h_ref, wg_ref, wa_ref, wb_ref, xw_ref, ta_ref, tb_ref):
    hb = h_ref[...]
    xw_ref[...] = jnp.dot(hb, wg_ref[...], preferred_element_type=_f32)
    a = jnp.dot(hb, wa_ref[...], preferred_element_type=_f32)
    b = jnp.dot(hb, wb_ref[...], preferred_element_type=_f32)
    s = jnp.sum(hb, axis=1, keepdims=True)
    q = jnp.sum(hb * hb, axis=1, keepdims=True)
    lane = lax.broadcasted_iota(_i32, (1, DT - 64), 1)
    stats = jnp.where(lane == 0, s, 0.0) + jnp.where(lane == 1, q, 0.0)
    ta_ref[...] = jnp.concatenate([a, stats], axis=1)
    tb_ref[...] = jnp.concatenate([b, stats], axis=1)


def _tc_pre(h, Wg, WA, WB):
    return pl.pallas_call(
        _tc_pre_body,
        grid=(NP // _BN,),
        in_specs=[
            pl.BlockSpec((_BN, C), lambda i: (i, 0)),
            pl.BlockSpec((C, HD), lambda i: (0, 0)),
            pl.BlockSpec((C, 64), lambda i: (0, 0)),
            pl.BlockSpec((C, 64), lambda i: (0, 0)),
        ],
        out_specs=[
            pl.BlockSpec((_BN, HD), lambda i: (i, 0)),
            pl.BlockSpec((_BN, DT), lambda i: (i, 0)),
            pl.BlockSpec((_BN, DT), lambda i: (i, 0)),
        ],
        out_shape=[
            jax.ShapeDtypeStruct((NP, HD), _f32),
            jax.ShapeDtypeStruct((NP, DT), _f32),
            jax.ShapeDtypeStruct((NP, DT), _f32),
        ],
    )(h, Wg, WA, WB)


def _tc_mid_body(dg_ref, xw_ref, y_ref):
    dis = lax.rsqrt(dg_ref[...])
    y_ref[...] = xw_ref[...] * dis


def _tc_mid(degsum, xw):
    return pl.pallas_call(
        _tc_mid_body,
        grid=(NP // _BN,),
        in_specs=[
            pl.BlockSpec((_BN, 1), lambda i: (i, 0)),
            pl.BlockSpec((_BN, HD), lambda i: (i, 0)),
        ],
        out_specs=pl.BlockSpec((_BN, HD), lambda i: (i, 0)),
        out_shape=jax.ShapeDtypeStruct((NP, HD), _f32),
    )(degsum, xw)


def _tc_post_body(dg_ref, a0_ref, a1_ref, xw_ref, bg_ref, o_ref, *, relu):
    dis = lax.rsqrt(dg_ref[...])
    o = (a0_ref[...] + a1_ref[...]) * dis + xw_ref[...] * (dis * dis) + bg_ref[...]
    if relu:
        o = jnp.maximum(o, 0.0)
    o_ref[...] = o


def _tc_post(degsum, acc0, acc1, xw, bg, relu):
    return pl.pallas_call(
        functools.partial(_tc_post_body, relu=relu),
        grid=(NP // _BN,),
        in_specs=[
            pl.BlockSpec((_BN, 1), lambda i: (i, 0)),
            pl.BlockSpec((_BN, HD), lambda i: (i, 0)),
            pl.BlockSpec((_BN, HD), lambda i: (i, 0)),
            pl.BlockSpec((_BN, HD), lambda i: (i, 0)),
            pl.BlockSpec((1, HD), lambda i: (0, 0)),
        ],
        out_specs=pl.BlockSpec((_BN, HD), lambda i: (i, 0)),
        out_shape=jax.ShapeDtypeStruct((NP, HD), _f32),
    )(degsum, acc0, acc1, xw, bg)


def _sigm(v):
    return 1.0 / (1.0 + jnp.exp(-v))


def _tc_head_body(h_ref, f0_ref, f1_ref, wa_ref, ba_ref, w1_ref, b1_ref,
                  w2_ref, b2_ref, w3_ref, b3_ref, o_ref):
    f0 = f0_ref[...]
    f1 = f1_ref[...]
    wa = wa_ref[...]
    ba = ba_ref[...]
    a0 = _sigm(jnp.dot(f0, wa, preferred_element_type=_f32) + ba)
    a1 = _sigm(jnp.dot(f1, wa, preferred_element_type=_f32) + ba)
    m = jnp.maximum(a0, a1)
    e0 = jnp.exp(a0 - m)
    e1 = jnp.exp(a1 - m)
    inv = 1.0 / (e0 + e1)
    ms = (e0 * inv) * f0 + (e1 * inv) * f1
    hh = h_ref[...] + ms
    z = jnp.maximum(jnp.dot(hh, w1_ref[...], preferred_element_type=_f32) + b1_ref[...], 0.0)
    z = jnp.maximum(jnp.dot(z, w2_ref[...], preferred_element_type=_f32) + b2_ref[...], 0.0)
    z = jnp.dot(z, w3_ref[...], preferred_element_type=_f32) + b3_ref[...]
    o_ref[...] = _sigm(z)


def _tc_head(h, f0, f1, Wa, ba, Wp1, bp1, Wp2, bp2, Wp3, bp3):
    full = lambda shape: pl.BlockSpec(shape, lambda i: tuple(0 for _ in shape))
    return pl.pallas_call(
        _tc_head_body,
        grid=(NP // _BN,),
        in_specs=[
            pl.BlockSpec((_BN, HD), lambda i: (i, 0)),
            pl.BlockSpec((_BN, HD), lambda i: (i, 0)),
            pl.BlockSpec((_BN, HD), lambda i: (i, 0)),
            full((HD, 1)), full((1, 1)),
            full((HD, HD)), full((1, HD)),
            full((HD, 64)), full((1, 64)),
            full((64, 1)), full((1, 1)),
        ],
        out_specs=pl.BlockSpec((_BN, 1), lambda i: (i, 0)),
        out_shape=jax.ShapeDtypeStruct((NP, 1), _f32),
    )(h, f0, f1, Wa, ba, Wp1, bp1, Wp2, bp2, Wp3, bp3)


def _rsqrt16(a):
    i = plsc.bitcast(a, _i32)
    i = jnp.int32(0x5F3759DF) - lax.shift_right_logical(i, 1)
    y = plsc.bitcast(i, _f32)
    for _ in range(3):
        y = y * (1.5 - 0.5 * a * y * y)
    return y


def _sc_pass1_body(ta_hbm, tb_hbm, srcp_hbm, dstp_hbm, tp_hbm, cv_hbm,
                   deg_hbm, csrc_hbm, cdst_hbm, cnt_hbm,
                   srcv, dstv, tv, bufa, bufb, deg_l, csrc, cdst, cvv, cntb,
                   sema, semb):
    cid = lax.axis_index("c")
    sid = lax.axis_index("s")
    tid = sid * NC + cid

    pltpu.sync_copy(srcp_hbm.at[tid], srcv)
    pltpu.sync_copy(dstp_hbm.at[tid], dstv)
    pltpu.sync_copy(tp_hbm.at[tid], tv)
    pltpu.sync_copy(cv_hbm, cvv)

    z16 = jnp.zeros((16,), _f32)
    pad16 = jnp.full((16,), N, _i32)

    def _zdeg(i, carry):
        deg_l[pl.ds(i * 16, 16)] = z16
        return carry

    lax.fori_loop(0, NP // 16, _zdeg, 0)

    def _zpad(i, carry):
        csrc[pl.ds(i * 16, 16)] = pad16
        cdst[pl.ds(i * 16, 16)] = pad16
        return carry

    lax.fori_loop(0, (EPT_PAD + 16) // 16, _zpad, 0)

    lanes0 = lax.iota(_i32, 16)
    gv = [cvv[pl.ds(k * 16, 16)] for k in range(4)]
    c0v = [cvv[pl.ds(64 + k * 16, 16)] for k in range(4)]
    w2v = [cvv[pl.ds(128 + k * 16, 16)] for k in range(4)]
    bs2 = cvv[pl.ds(192, 16)][0]

    def _chunk(j, cnt):
        d1 = pltpu.async_copy(ta_hbm.at[srcv.at[j]], bufa, sema)
        d2 = pltpu.async_copy(tb_hbm.at[dstv.at[j]], bufb, semb)
        d1.wait()
        d2.wait()
        for g in range(CHW // 16):
            lanes = lanes0 + (g * 16)
            c64 = jnp.full((16,), 64, _i32)
            c65 = jnp.full((16,), 65, _i32)
            ssum = plsc.load_gather(bufa, [lanes, c64]) + plsc.load_gather(bufb, [lanes, c64])
            qsum = plsc.load_gather(bufa, [lanes, c65]) + plsc.load_gather(bufb, [lanes, c65])
            mu = ssum * (1.0 / (2 * C))
            a = qsum * (1.0 / (2 * C)) - mu * mu + 1e-5
            rs = _rsqrt16(a)
            m2 = mu * rs
            zacc = bs2 + z16
            for jj in range(64):
                k, m = divmod(jj, 16)
                cj = jnp.full((16,), jj, _i32)
                va = plsc.load_gather(bufa, [lanes, cj])
                vb = plsc.load_gather(bufb, [lanes, cj])
                v = (va + vb) * rs + (c0v[k][m] - m2 * gv[k][m])
                v = jnp.maximum(v, 0.0)
                zacc = zacc + v * w2v[k][m]
            tvec = tv[j, pl.ds(g * 16, 16)]
            mask = zacc > tvec
            svec = srcv[j, pl.ds(g * 16, 16)]
            dvec = dstv[j, pl.ds(g * 16, 16)]
            plsc.addupdate_scatter(deg_l, [dvec], jnp.where(mask, 1.0, 0.0))
            plsc.store_compressed(csrc.at[pl.ds(cnt, 16)], svec, mask=mask)
            plsc.store_compressed(cdst.at[pl.ds(cnt, 16)], dvec, mask=mask)
            cnt = cnt + jnp.sum(mask.astype(_i32))
        return cnt

    cnt = lax.fori_loop(0, NCHK, _chunk, jnp.int32(0))

    cvec16 = jnp.full((16,), 0, _i32) + cnt
    for kk in range(8):
        cntb[pl.ds(kk * 16, 16)] = cvec16
    pltpu.sync_copy(cntb, cnt_hbm.at[tid])
    pltpu.sync_copy(deg_l, deg_hbm.at[tid])
    pltpu.sync_copy(csrc.at[pl.ds(0, EPT_PAD)], csrc_hbm.at[tid])
    pltpu.sync_copy(cdst.at[pl.ds(0, EPT_PAD)], cdst_hbm.at[tid])


def _sc_pass1(ta, tb, srcp, dstp, tp, cvec):
    mesh = plsc.VectorSubcoreMesh(core_axis_name="c", subcore_axis_name="s")
    return pl.kernel(
        _sc_pass1_body,
        out_type=[
            jax.ShapeDtypeStruct((NW, NP), _f32),
            jax.ShapeDtypeStruct((NW, EPT_PAD), _i32),
            jax.ShapeDtypeStruct((NW, EPT_PAD), _i32),
            jax.ShapeDtypeStruct((NW, 128), _i32),
        ],
        mesh=mesh,
        compiler_params=pltpu.CompilerParams(needs_layout_passes=False),
        scratch_types=[
            pltpu.VMEM((NCHK, CHW), _i32),
            pltpu.VMEM((NCHK, CHW), _i32),
            pltpu.VMEM((NCHK, CHW), _f32),
            pltpu.VMEM((CHW, DT), _f32),
            pltpu.VMEM((CHW, DT), _f32),
            pltpu.VMEM((NP,), _f32),
            pltpu.VMEM((EPT_PAD + 16,), _i32),
            pltpu.VMEM((EPT_PAD + 16,), _i32),
            pltpu.VMEM((256,), _f32),
            pltpu.VMEM((128,), _i32),
            pltpu.SemaphoreType.DMA,
            pltpu.SemaphoreType.DMA,
        ],
    )(ta, tb, srcp, dstp, tp, cvec)


def _sc_pass2_body(y_hbm, csrc_hbm, cdst_hbm, cnt_hbm, accs_hbm,
                   srcv, dstv, buf, cntb, acc, sem):
    cid = lax.axis_index("c")
    sid = lax.axis_index("s")
    tid = sid * NC + cid
    rows_per_tile = NP // NS

    z16 = jnp.zeros((16,), _f32)

    def _zrow(i, carry):
        for k in range(HD // 16):
            buf[i, pl.ds(k * 16, 16)] = z16
        return carry

    lax.fori_loop(0, CHW, _zrow, 0)
    for i in range(rows_per_tile // CHW):
        pltpu.sync_copy(buf, acc.at[pl.ds(sid * rows_per_tile + i * CHW, CHW)])
    plsc.subcore_barrier()

    pltpu.sync_copy(csrc_hbm.at[tid], srcv)
    pltpu.sync_copy(cdst_hbm.at[tid], dstv)
    pltpu.sync_copy(cnt_hbm.at[tid], cntb)
    n = cntb[pl.ds(0, 16)][0]
    nch = (n + (CHW - 1)) // CHW

    def _body(j, carry):
        pltpu.async_copy(y_hbm.at[srcv.at[j]], buf, sem).wait()
        pltpu.sync_copy(buf, acc.at[dstv.at[j]], add=True)
        return carry

    lax.fori_loop(0, nch, _body, 0)
    plsc.subcore_barrier()
    for i in range(rows_per_tile // CHW):
        r0 = sid * rows_per_tile + i * CHW
        pltpu.sync_copy(acc.at[pl.ds(r0, CHW)], accs_hbm.at[cid, pl.ds(r0, CHW)])


def _sc_pass2(y, csrc, cdst, cnt):
    mesh = plsc.VectorSubcoreMesh(core_axis_name="c", subcore_axis_name="s")
    return pl.kernel(
        _sc_pass2_body,
        out_type=jax.ShapeDtypeStruct((NC, NP, HD), _f32),
        mesh=mesh,
        compiler_params=pltpu.CompilerParams(needs_layout_passes=False),
        scratch_types=[
            pltpu.VMEM((NCHK, CHW), _i32),
            pltpu.VMEM((NCHK, CHW), _i32),
            pltpu.VMEM((CHW, HD), _f32),
            pltpu.VMEM((128,), _i32),
            pltpu.VMEM_SHARED((NP, HD), _f32),
            pltpu.SemaphoreType.DMA,
        ],
    )(y, csrc, cdst, cnt)


def kernel(x, edge_index, Wg0, bg0, lng0, lnb0, Ws1_0, bs1_0, Ws2_0, bs2_0,
           Wg1, bg1, lng1, lnb1, Ws1_1, bs1_1, Ws2_1, bs2_1,
           Wg2, bg2, lng2, lnb2, Ws1_2, bs1_2, Ws2_2, bs2_2,
           Wa, ba, Wp1, bp1, Wp2, bp2, Wp3, bp3):
    layers = [
        (Wg0, bg0, lng0, lnb0, Ws1_0, bs1_0, Ws2_0, bs2_0),
        (Wg1, bg1, lng1, lnb1, Ws1_1, bs1_1, Ws2_1, bs2_1),
        (Wg2, bg2, lng2, lnb2, Ws1_2, bs1_2, Ws2_2, bs2_2),
    ]
    src = edge_index[0]
    dst = edge_index[1]
    srcp = jnp.pad(src.reshape(NW, EPT), ((0, 0), (0, EPT_PAD - EPT)),
                   constant_values=N).reshape(NW, NCHK, CHW)
    dstp = jnp.pad(dst.reshape(NW, EPT), ((0, 0), (0, EPT_PAD - EPT)),
                   constant_values=N).reshape(NW, NCHK, CHW)

    base = jax.random.key(42)
    h = jnp.pad(x, ((0, NP - N), (0, 0)))
    feats = []
    for l, (Wg, bg, lng, lnb, Ws1, bs1, Ws2, bs2) in enumerate(layers):
        u = jax.random.uniform(jax.random.fold_in(base, l), (E,), _f32)
        t = jnp.log(u) - jnp.log1p(-u)
        tp = jnp.pad(t.reshape(NW, EPT), ((0, 0), (0, EPT_PAD - EPT)),
                     constant_values=jnp.inf).reshape(NW, NCHK, CHW)
        WA = lng[:C, None] * Ws1[:C]
        WB = lng[C:, None] * Ws1[C:]
        g64 = lng @ Ws1
        c0 = lnb @ Ws1 + bs1
        cvec = jnp.concatenate([g64, c0, Ws2[:, 0], jnp.full((64,), bs2[0], _f32)])

        xw, ta, tb = _tc_pre(h, Wg, WA, WB)
        deg, csrc, cdst, cnt = _sc_pass1(ta, tb, srcp, dstp, tp, cvec)
        degsum = (deg.sum(axis=0) + 1.0).reshape(NP, 1)
        y = _tc_mid(degsum, xw)
        accs = _sc_pass2(y, csrc.reshape(NW, NCHK, CHW), cdst.reshape(NW, NCHK, CHW), cnt)
        h = _tc_post(degsum, accs[0], accs[1], xw, bg.reshape(1, HD), relu=(l != 2))
        if l != 2:
            feats.append(h)

    out = _tc_head(h, feats[0], feats[1], Wa.reshape(HD, 1), ba.reshape(1, 1),
                   Wp1, bp1.reshape(1, HD), Wp2, bp2.reshape(1, 64),
                   Wp3, bp3.reshape(1, 1))
    return out[:N]


# speedup vs baseline: 12.5460x; 1.6693x over previous
import functools

import jax
import jax.numpy as jnp
from jax import lax
from jax.experimental import pallas as pl
from jax.experimental.pallas import tpu as pltpu
from jax.experimental.pallas import tpu_sc as plsc

N = 10000
NP = 10240
E = 160000
C = 128
HD = 128
DT = 128
NC = 2
NS = 16
NW = NC * NS
EPT = E // NW
CHW = 128
NCHK = 40
EPT_PAD = NCHK * CHW

_f32 = jnp.float32
_i32 = jnp.int32


_BN = 1024


def _tc_pre_body(h_ref, wg_ref, wa_ref, wb_ref, xw_ref, ta_ref, tb_ref):
    hb = h_ref[...]
    xw_ref[...] = jnp.dot(hb, wg_ref[...], preferred_element_type=_f32)
    a = jnp.dot(hb, wa_ref[...], preferred_element_type=_f32)
    b = jnp.dot(hb, wb_ref[...], preferred_element_type=_f32)
    s = jnp.sum(hb, axis=1, keepdims=True)
    q = jnp.sum(hb * hb, axis=1, keepdims=True)
    lane = lax.broadcasted_iota(_i32, (1, DT - 64), 1)
    stats = jnp.where(lane == 0, s, 0.0) + jnp.where(lane == 1, q, 0.0)
    ta_ref[...] = jnp.concatenate([a, stats], axis=1)
    tb_ref[...] = jnp.concatenate([b, stats], axis=1)


def _tc_pre(h, Wg, WA, WB):
    return pl.pallas_call(
        _tc_pre_body,
        grid=(NP // _BN,),
        in_specs=[
            pl.BlockSpec((_BN, C), lambda i: (i, 0)),
            pl.BlockSpec((C, HD), lambda i: (0, 0)),
            pl.BlockSpec((C, 64), lambda i: (0, 0)),
            pl.BlockSpec((C, 64), lambda i: (0, 0)),
        ],
        out_specs=[
            pl.BlockSpec((_BN, HD), lambda i: (i, 0)),
            pl.BlockSpec((_BN, DT), lambda i: (i, 0)),
            pl.BlockSpec((_BN, DT), lambda i: (i, 0)),
        ],
        out_shape=[
            jax.ShapeDtypeStruct((NP, HD), _f32),
            jax.ShapeDtypeStruct((NP, DT), _f32),
            jax.ShapeDtypeStruct((NP, DT), _f32),
        ],
    )(h, Wg, WA, WB)


def _tc_mid_body(dg_ref, xw_ref, y_ref):
    dis = lax.rsqrt(dg_ref[...])
    y_ref[...] = xw_ref[...] * dis


def _tc_mid(degsum, xw):
    return pl.pallas_call(
        _tc_mid_body,
        grid=(NP // _BN,),
        in_specs=[
            pl.BlockSpec((_BN, 1), lambda i: (i, 0)),
            pl.BlockSpec((_BN, HD), lambda i: (i, 0)),
        ],
        out_specs=pl.BlockSpec((_BN, HD), lambda i: (i, 0)),
        out_shape=jax.ShapeDtypeStruct((NP, HD), _f32),
    )(degsum, xw)


def _tc_post_body(dg_ref, a0_ref, a1_ref, xw_ref, bg_ref, o_ref, *, relu):
    dis = lax.rsqrt(dg_ref[...])
    o = (a0_ref[...] + a1_ref[...]) * dis + xw_ref[...] * (dis * dis) + bg_ref[...]
    if relu:
        o = jnp.maximum(o, 0.0)
    o_ref[...] = o


def _tc_post(degsum, acc0, acc1, xw, bg, relu):
    return pl.pallas_call(
        functools.partial(_tc_post_body, relu=relu),
        grid=(NP // _BN,),
        in_specs=[
            pl.BlockSpec((_BN, 1), lambda i: (i, 0)),
            pl.BlockSpec((_BN, HD), lambda i: (i, 0)),
            pl.BlockSpec((_BN, HD), lambda i: (i, 0)),
            pl.BlockSpec((_BN, HD), lambda i: (i, 0)),
            pl.BlockSpec((1, HD), lambda i: (0, 0)),
        ],
        out_specs=pl.BlockSpec((_BN, HD), lambda i: (i, 0)),
        out_shape=jax.ShapeDtypeStruct((NP, HD), _f32),
    )(degsum, acc0, acc1, xw, bg)


def _sigm(v):
    return 1.0 / (1.0 + jnp.exp(-v))


def _tc_head_body(h_ref, f0_ref, f1_ref, wa_ref, ba_ref, w1_ref, b1_ref,
                  w2_ref, b2_ref, w3_ref, b3_ref, o_ref):
    f0 = f0_ref[...]
    f1 = f1_ref[...]
    wa = wa_ref[...]
    ba = ba_ref[...]
    a0 = _sigm(jnp.dot(f0, wa, preferred_element_type=_f32) + ba)
    a1 = _sigm(jnp.dot(f1, wa, preferred_element_type=_f32) + ba)
    m = jnp.maximum(a0, a1)
    e0 = jnp.exp(a0 - m)
    e1 = jnp.exp(a1 - m)
    inv = 1.0 / (e0 + e1)
    ms = (e0 * inv) * f0 + (e1 * inv) * f1
    hh = h_ref[...] + ms
    z = jnp.maximum(jnp.dot(hh, w1_ref[...], preferred_element_type=_f32) + b1_ref[...], 0.0)
    z = jnp.maximum(jnp.dot(z, w2_ref[...], preferred_element_type=_f32) + b2_ref[...], 0.0)
    z = jnp.dot(z, w3_ref[...], preferred_element_type=_f32) + b3_ref[...]
    o_ref[...] = _sigm(z)


def _tc_head(h, f0, f1, Wa, ba, Wp1, bp1, Wp2, bp2, Wp3, bp3):
    full = lambda shape: pl.BlockSpec(shape, lambda i: tuple(0 for _ in shape))
    return pl.pallas_call(
        _tc_head_body,
        grid=(NP // _BN,),
        in_specs=[
            pl.BlockSpec((_BN, HD), lambda i: (i, 0)),
            pl.BlockSpec((_BN, HD), lambda i: (i, 0)),
            pl.BlockSpec((_BN, HD), lambda i: (i, 0)),
            full((HD, 1)), full((1, 1)),
            full((HD, HD)), full((1, HD)),
            full((HD, 64)), full((1, 64)),
            full((64, 1)), full((1, 1)),
        ],
        out_specs=pl.BlockSpec((_BN, 1), lambda i: (i, 0)),
        out_shape=jax.ShapeDtypeStruct((NP, 1), _f32),
    )(h, f0, f1, Wa, ba, Wp1, bp1, Wp2, bp2, Wp3, bp3)


def _rsqrt16(a):
    i = plsc.bitcast(a, _i32)
    i = jnp.int32(0x5F3759DF) - lax.shift_right_logical(i, 1)
    y = plsc.bitcast(i, _f32)
    for _ in range(3):
        y = y * (1.5 - 0.5 * a * y * y)
    return y


def _sc_pass1_body(ta_hbm, tb_hbm, srcp_hbm, dstp_hbm, tp_hbm, cv_hbm,
                   deg_hbm, csrc_hbm, cdst_hbm, cnt_hbm,
                   srcv, dstv, tv, bufa, bufb, deg_l, csrc, cdst, cvv, cntb,
                   sema, semb):
    cid = lax.axis_index("c")
    sid = lax.axis_index("s")
    tid = sid * NC + cid

    pltpu.sync_copy(srcp_hbm.at[tid], srcv)
    pltpu.sync_copy(dstp_hbm.at[tid], dstv)
    pltpu.sync_copy(tp_hbm.at[tid], tv)
    pltpu.sync_copy(cv_hbm, cvv)

    z16 = jnp.zeros((16,), _f32)
    pad16 = jnp.full((16,), N, _i32)

    def _zdeg(i, carry):
        deg_l[pl.ds(i * 16, 16)] = z16
        return carry

    lax.fori_loop(0, NP // 16, _zdeg, 0)

    def _zpad(i, carry):
        csrc[pl.ds(i * 16, 16)] = pad16
        cdst[pl.ds(i * 16, 16)] = pad16
        return carry

    lax.fori_loop(0, (EPT_PAD + 16) // 16, _zpad, 0)

    lanes0 = lax.iota(_i32, 16)
    gv = [cvv[pl.ds(k * 16, 16)] for k in range(4)]
    c0v = [cvv[pl.ds(64 + k * 16, 16)] for k in range(4)]
    w2v = [cvv[pl.ds(128 + k * 16, 16)] for k in range(4)]
    bs2 = cvv[pl.ds(192, 16)][0]

    def _chunk(j, cnt):
        d1 = pltpu.async_copy(ta_hbm.at[srcv.at[j]], bufa, sema)
        d2 = pltpu.async_copy(tb_hbm.at[dstv.at[j]], bufb, semb)
        d1.wait()
        d2.wait()

        def _group(g, cnt):
            gb = g * 16
            lanes = lanes0 + gb
            c64 = jnp.full((16,), 64, _i32)
            c65 = jnp.full((16,), 65, _i32)
            ssum = plsc.load_gather(bufa, [lanes, c64]) + plsc.load_gather(bufb, [lanes, c64])
            qsum = plsc.load_gather(bufa, [lanes, c65]) + plsc.load_gather(bufb, [lanes, c65])
            mu = ssum * (1.0 / (2 * C))
            a = qsum * (1.0 / (2 * C)) - mu * mu + 1e-5
            rs16 = _rsqrt16(a)
            m216 = mu * rs16
            zv = z16
            for i in range(16):
                e = gb + i
                rs_b = z16 + rs16[i]
                m2_b = z16 + m216[i]
                facc = z16
                for k in range(4):
                    va = bufa[e, pl.ds(k * 16, 16)]
                    vb = bufb[e, pl.ds(k * 16, 16)]
                    v = (va + vb) * rs_b + (c0v[k] - m2_b * gv[k])
                    v = jnp.maximum(v, 0.0)
                    facc = facc + v * w2v[k]
                z_e = jnp.sum(facc) + bs2
                zv = jnp.where(lanes0 == i, z_e, zv)
            tvec = tv[j, pl.ds(gb, 16)]
            mask = zv > tvec
            svec = srcv[j, pl.ds(gb, 16)]
            dvec = dstv[j, pl.ds(gb, 16)]
            plsc.addupdate_scatter(deg_l, [dvec], jnp.where(mask, 1.0, 0.0))
            plsc.store_compressed(csrc.at[pl.ds(cnt, 16)], svec, mask=mask)
            plsc.store_compressed(cdst.at[pl.ds(cnt, 16)], dvec, mask=mask)
            return cnt + jnp.sum(mask.astype(_i32))

        return lax.fori_loop(0, CHW // 16, _group, cnt)

    cnt = lax.fori_loop(0, NCHK, _chunk, jnp.int32(0))

    cvec16 = jnp.full((16,), 0, _i32) + cnt
    for kk in range(8):
        cntb[pl.ds(kk * 16, 16)] = cvec16
    pltpu.sync_copy(cntb, cnt_hbm.at[tid])
    pltpu.sync_copy(deg_l, deg_hbm.at[tid])
    pltpu.sync_copy(csrc.at[pl.ds(0, EPT_PAD)], csrc_hbm.at[tid])
    pltpu.sync_copy(cdst.at[pl.ds(0, EPT_PAD)], cdst_hbm.at[tid])


def _sc_pass1(ta, tb, srcp, dstp, tp, cvec):
    mesh = plsc.VectorSubcoreMesh(core_axis_name="c", subcore_axis_name="s")
    return pl.kernel(
        _sc_pass1_body,
        out_type=[
            jax.ShapeDtypeStruct((NW, NP), _f32),
            jax.ShapeDtypeStruct((NW, EPT_PAD), _i32),
            jax.ShapeDtypeStruct((NW, EPT_PAD), _i32),
            jax.ShapeDtypeStruct((NW, 128), _i32),
        ],
        mesh=mesh,
        compiler_params=pltpu.CompilerParams(needs_layout_passes=False),
        scratch_types=[
            pltpu.VMEM((NCHK, CHW), _i32),
            pltpu.VMEM((NCHK, CHW), _i32),
            pltpu.VMEM((NCHK, CHW), _f32),
            pltpu.VMEM((CHW, DT), _f32),
            pltpu.VMEM((CHW, DT), _f32),
            pltpu.VMEM((NP,), _f32),
            pltpu.VMEM((EPT_PAD + 16,), _i32),
            pltpu.VMEM((EPT_PAD + 16,), _i32),
            pltpu.VMEM((256,), _f32),
            pltpu.VMEM((128,), _i32),
            pltpu.SemaphoreType.DMA,
            pltpu.SemaphoreType.DMA,
        ],
    )(ta, tb, srcp, dstp, tp, cvec)


def _sc_pass2_body(y_hbm, csrc_hbm, cdst_hbm, cnt_hbm, accs_hbm,
                   srcv, dstv, buf, cntb, acc, sem):
    cid = lax.axis_index("c")
    sid = lax.axis_index("s")
    tid = sid * NC + cid
    rows_per_tile = NP // NS

    z16 = jnp.zeros((16,), _f32)

    def _zrow(i, carry):
        for k in range(HD // 16):
            buf[i, pl.ds(k * 16, 16)] = z16
        return carry

    lax.fori_loop(0, CHW, _zrow, 0)
    for i in range(rows_per_tile // CHW):
        pltpu.sync_copy(buf, acc.at[pl.ds(sid * rows_per_tile + i * CHW, CHW)])
    plsc.subcore_barrier()

    pltpu.sync_copy(csrc_hbm.at[tid], srcv)
    pltpu.sync_copy(cdst_hbm.at[tid], dstv)
    pltpu.sync_copy(cnt_hbm.at[tid], cntb)
    n = cntb[pl.ds(0, 16)][0]
    nch = (n + (CHW - 1)) // CHW

    def _body(j, carry):
        pltpu.async_copy(y_hbm.at[srcv.at[j]], buf, sem).wait()
        pltpu.sync_copy(buf, acc.at[dstv.at[j]], add=True)
        return carry

    lax.fori_loop(0, nch, _body, 0)
    plsc.subcore_barrier()
    for i in range(rows_per_tile // CHW):
        r0 = sid * rows_per_tile + i * CHW
        pltpu.sync_copy(acc.at[pl.ds(r0, CHW)], accs_hbm.at[cid, pl.ds(r0, CHW)])


def _sc_pass2(y, csrc, cdst, cnt):
    mesh = plsc.VectorSubcoreMesh(core_axis_name="c", subcore_axis_name="s")
    return pl.kernel(
        _sc_pass2_body,
        out_type=jax.ShapeDtypeStruct((NC, NP, HD), _f32),
        mesh=mesh,
        compiler_params=pltpu.CompilerParams(needs_layout_passes=False),
        scratch_types=[
            pltpu.VMEM((NCHK, CHW), _i32),
            pltpu.VMEM((NCHK, CHW), _i32),
            pltpu.VMEM((CHW, HD), _f32),
            pltpu.VMEM((128,), _i32),
            pltpu.VMEM_SHARED((NP, HD), _f32),
            pltpu.SemaphoreType.DMA,
        ],
    )(y, csrc, cdst, cnt)


def kernel(x, edge_index, Wg0, bg0, lng0, lnb0, Ws1_0, bs1_0, Ws2_0, bs2_0,
           Wg1, bg1, lng1, lnb1, Ws1_1, bs1_1, Ws2_1, bs2_1,
           Wg2, bg2, lng2, lnb2, Ws1_2, bs1_2, Ws2_2, bs2_2,
           Wa, ba, Wp1, bp1, Wp2, bp2, Wp3, bp3):
    layers = [
        (Wg0, bg0, lng0, lnb0, Ws1_0, bs1_0, Ws2_0, bs2_0),
        (Wg1, bg1, lng1, lnb1, Ws1_1, bs1_1, Ws2_1, bs2_1),
        (Wg2, bg2, lng2, lnb2, Ws1_2, bs1_2, Ws2_2, bs2_2),
    ]
    src = edge_index[0]
    dst = edge_index[1]
    srcp = jnp.pad(src.reshape(NW, EPT), ((0, 0), (0, EPT_PAD - EPT)),
                   constant_values=N).reshape(NW, NCHK, CHW)
    dstp = jnp.pad(dst.reshape(NW, EPT), ((0, 0), (0, EPT_PAD - EPT)),
                   constant_values=N).reshape(NW, NCHK, CHW)

    base = jax.random.key(42)
    h = jnp.pad(x, ((0, NP - N), (0, 0)))
    feats = []
    for l, (Wg, bg, lng, lnb, Ws1, bs1, Ws2, bs2) in enumerate(layers):
        u = jax.random.uniform(jax.random.fold_in(base, l), (E,), _f32)
        t = jnp.log(u) - jnp.log1p(-u)
        tp = jnp.pad(t.reshape(NW, EPT), ((0, 0), (0, EPT_PAD - EPT)),
                     constant_values=jnp.inf).reshape(NW, NCHK, CHW)
        WA = lng[:C, None] * Ws1[:C]
        WB = lng[C:, None] * Ws1[C:]
        g64 = lng @ Ws1
        c0 = lnb @ Ws1 + bs1
        cvec = jnp.concatenate([g64, c0, Ws2[:, 0], jnp.full((64,), bs2[0], _f32)])

        xw, ta, tb = _tc_pre(h, Wg, WA, WB)
        deg, csrc, cdst, cnt = _sc_pass1(ta, tb, srcp, dstp, tp, cvec)
        degsum = (deg.sum(axis=0) + 1.0).reshape(NP, 1)
        y = _tc_mid(degsum, xw)
        accs = _sc_pass2(y, csrc.reshape(NW, NCHK, CHW), cdst.reshape(NW, NCHK, CHW), cnt)
        h = _tc_post(degsum, accs[0], accs[1], xw, bg.reshape(1, HD), relu=(l != 2))
        if l != 2:
            feats.append(h)

    out = _tc_head(h, feats[0], feats[1], Wa.reshape(HD, 1), ba.reshape(1, 1),
                   Wp1, bp1.reshape(1, HD), Wp2, bp2.reshape(1, 64),
                   Wp3, bp3.reshape(1, 1))
    return out[:N]
